# baseline (speedup 1.0000x reference)
import jax
import jax.numpy as jnp
from jax import lax
from jax.experimental import pallas as pl
from jax.experimental.pallas import tpu as pltpu
from jax.experimental.pallas import tpu_sc as plsc

N_OUT = 4096
N_CMD = 4
BATCH = 4096
N_IN = N_OUT * N_CMD

NC = 2
NS = 16
L = 16
NW = NC * NS
ROWS_PER_W = BATCH // NW
KR = 2
NCHUNK = ROWS_PER_W // KR


def _sc_body(x_hbm, cmd_hbm, out_hbm, in0, in1, out0, out1, cmd_v,
             si0, si1, so0, so1):
    wid = lax.axis_index("s") * NC + lax.axis_index("c")
    ins, outs = (in0, in1), (out0, out1)
    sis, sos = (si0, si1), (so0, so1)

    pltpu.sync_copy(cmd_hbm, cmd_v)
    cvec = cmd_v[...]
    colbase = lax.iota(jnp.int32, L) * N_CMD + cvec

    in_base = wid * (ROWS_PER_W * N_IN)
    out_base = wid * (ROWS_PER_W * N_OUT)

    def start_in(ch, b):
        pltpu.async_copy(
            x_hbm.at[pl.ds(in_base + ch * (KR * N_IN), KR * N_IN)],
            ins[b], sis[b])

    def wait_in(b):
        pltpu.make_async_copy(
            x_hbm.at[pl.ds(0, KR * N_IN)], ins[b], sis[b]).wait()

    def start_out(ch, b):
        pltpu.async_copy(
            outs[b],
            out_hbm.at[pl.ds(out_base + ch * (KR * N_OUT), KR * N_OUT)],
            sos[b])

    def wait_out(b):
        pltpu.make_async_copy(
            outs[b], out_hbm.at[pl.ds(0, KR * N_OUT)], sos[b]).wait()

    start_in(0, 0)

    def gbody(g, carry):
        for b in range(2):
            ch = g * 2 + b

            @pl.when(ch + 1 < NCHUNK)
            def _():
                start_in(ch + 1, 1 - b)

            wait_in(b)

            @pl.when(ch >= 2)
            def _():
                wait_out(b)

            for rr in range(KR):
                rbase_in = rr * N_IN
                rbase_out = rr * N_OUT

                def jbody(j, c2, _rin=rbase_in, _rout=rbase_out, _b=b):
                    col = _rin + j * (N_CMD * L) + colbase
                    vals = plsc.load_gather(ins[_b], [col])
                    outs[_b][pl.ds(_rout + j * L, L)] = vals
                    return c2

                lax.fori_loop(0, N_OUT // L, jbody, 0, unroll=8)
            start_out(ch, b)
        return carry

    lax.fori_loop(0, NCHUNK // 2, gbody, 0)
    wait_out(0)
    wait_out(1)


@jax.jit
def kernel(x, command):
    cmd16 = jnp.broadcast_to(command.astype(jnp.int32), (L,))
    x1 = x.reshape(-1)
    mesh = plsc.VectorSubcoreMesh(core_axis_name="c", subcore_axis_name="s")
    run = pl.kernel(
        _sc_body,
        out_type=jax.ShapeDtypeStruct((BATCH * N_OUT,), jnp.float32),
        mesh=mesh,
        scratch_types=[
            pltpu.VMEM((KR * N_IN,), jnp.float32),
            pltpu.VMEM((KR * N_IN,), jnp.float32),
            pltpu.VMEM((KR * N_OUT,), jnp.float32),
            pltpu.VMEM((KR * N_OUT,), jnp.float32),
            pltpu.VMEM((L,), jnp.int32),
            pltpu.SemaphoreType.DMA,
            pltpu.SemaphoreType.DMA,
            pltpu.SemaphoreType.DMA,
            pltpu.SemaphoreType.DMA,
        ],
        compiler_params=pltpu.CompilerParams(needs_layout_passes=False),
    )
    return run(x1, cmd16).reshape(BATCH, N_OUT)


# speedup vs baseline: 3.9777x; 3.9777x over previous
import jax
import jax.numpy as jnp
from jax.experimental import pallas as pl
from jax.experimental.pallas import tpu as pltpu

N_OUT = 4096
N_CMD = 4
BATCH = 4096

BB = 128
KCH = 512
NCH = KCH // N_CMD


def _tc_body(cmd_ref, x_ref, o_ref):
    c = cmd_ref[0]
    j = jax.lax.broadcasted_iota(jnp.int32, (KCH, NCH), 0)
    n = jax.lax.broadcasted_iota(jnp.int32, (KCH, NCH), 1)
    s = (j == N_CMD * n + c).astype(jnp.float32)
    for g in range(N_OUT * N_CMD // KCH):
        o_ref[:, g * NCH:(g + 1) * NCH] = jnp.dot(
            x_ref[:, g * KCH:(g + 1) * KCH], s,
            preferred_element_type=jnp.float32)


@jax.jit
def kernel(x, command):
    grid_spec = pltpu.PrefetchScalarGridSpec(
        num_scalar_prefetch=1,
        grid=(BATCH // BB,),
        in_specs=[pl.BlockSpec((BB, N_OUT * N_CMD), lambda i, c: (i, 0))],
        out_specs=pl.BlockSpec((BB, N_OUT), lambda i, c: (i, 0)),
    )
    return pl.pallas_call(
        _tc_body,
        grid_spec=grid_spec,
        out_shape=jax.ShapeDtypeStruct((BATCH, N_OUT), jnp.float32),
    )(command, x)


# speedup vs baseline: 3.9925x; 1.0037x over previous
import jax
import jax.numpy as jnp
from jax.experimental import pallas as pl
from jax.experimental.pallas import tpu as pltpu

N_OUT = 4096
N_CMD = 4
BATCH = 4096

BB = 128
KCH = 512
NCH = KCH // N_CMD


def _tc_body(cmd_ref, x_ref, o_ref):
    c = cmd_ref[0]
    j = jax.lax.broadcasted_iota(jnp.int32, (KCH, NCH), 0)
    n = jax.lax.broadcasted_iota(jnp.int32, (KCH, NCH), 1)
    s = (j == N_CMD * n + c).astype(jnp.float32)
    for g in range(N_OUT * N_CMD // KCH):
        o_ref[:, g * NCH:(g + 1) * NCH] = jnp.dot(
            x_ref[:, g * KCH:(g + 1) * KCH], s,
            preferred_element_type=jnp.float32)


@jax.jit
def kernel(x, command):
    grid_spec = pltpu.PrefetchScalarGridSpec(
        num_scalar_prefetch=1,
        grid=(BATCH // BB,),
        in_specs=[pl.BlockSpec((BB, N_OUT * N_CMD), lambda i, c: (i, 0))],
        out_specs=pl.BlockSpec((BB, N_OUT), lambda i, c: (i, 0)),
    )
    return pl.pallas_call(
        _tc_body,
        grid_spec=grid_spec,
        out_shape=jax.ShapeDtypeStruct((BATCH, N_OUT), jnp.float32),
        compiler_params=pltpu.CompilerParams(
            dimension_semantics=("parallel",)),
    )(command, x)
